# baseline (speedup 1.0000x reference)
import jax
import jax.numpy as jnp
from jax import lax
from jax.experimental import pallas as pl
from jax.experimental.pallas import tpu as pltpu
from jax.experimental.pallas import tpu_sc as plsc

C = 1203
D = 256
N = 65536
TEMP = 0.07
CP = 1280
NC, NS = 2, 16
NW = NC * NS
ND = 4
DSUB = D // ND
NR = NW // ND
ROWS_R = N // NR
CHUNK = 64
NCH = ROWS_R // CHUNK
NEG = -1e30


def _sc_body(feats_hbm, labels_hbm, sums_hbm, cnt_hbm,
             labels_v, f0, f1, acc_v, cnt_v, sem0, sem1):
    cid = lax.axis_index("c")
    sid = lax.axis_index("s")
    wid = cid * NS + sid
    r = wid // ND
    d = wid % ND
    h = d // 2
    off = (d % 2) * DSUB
    row0 = r * ROWS_R

    zero16 = jnp.zeros((16,), jnp.float32)
    ones16 = jnp.ones((16,), jnp.float32)
    lane = lax.iota(jnp.int32, 16)
    cols = [lane + k * 16 for k in range(DSUB // 16)]

    @plsc.parallel_loop(0, CP * DSUB // 16, 1, unroll=4)
    def _zero(i):
        acc_v[pl.ds(i * 16, 16)] = zero16

    @plsc.parallel_loop(0, CP, 1, unroll=4)
    def _zeroc(i):
        cnt_v[pl.ds(i * 16, 16)] = zero16

    pltpu.sync_copy(labels_hbm.at[pl.ds(row0, ROWS_R)], labels_v)

    def dma_start(j, buf, sem):
        pltpu.async_copy(
            feats_hbm.at[pl.ds(row0 + j * CHUNK, CHUNK), pl.ds(h * 128, 128)],
            buf, sem)

    def dma_wait(buf, sem):
        pltpu.make_async_copy(
            feats_hbm.at[pl.ds(row0, CHUNK), pl.ds(h * 128, 128)],
            buf, sem).wait()

    def compute(j, buf):
        jbase = j * CHUNK

        @pl.when(d == 0)
        def _():
            @plsc.parallel_loop(0, CHUNK // 16, 1, unroll=2)
            def _cnt16(t):
                lv = labels_v[pl.ds(jbase + t * 16, 16)]
                plsc.addupdate_scatter(cnt_v, [lv * 16 + lane], ones16)

        @plsc.parallel_loop(0, CHUNK, 1, unroll=4)
        def _rows(i):
            ridx = jnp.full((16,), jbase + i, jnp.int32)
            bl = plsc.load_gather(labels_v, [ridx]) * DSUB
            for k in range(DSUB // 16):
                v = buf[i, pl.ds(off + k * 16, 16)]
                plsc.addupdate_scatter(acc_v, [bl + cols[k]], v)

    dma_start(0, f0, sem0)

    def outer(j2, c):
        j = j2 * 2
        dma_wait(f0, sem0)
        dma_start(jnp.minimum(j + 1, NCH - 1), f1, sem1)
        compute(j, f0)
        dma_wait(f1, sem1)
        dma_start(jnp.minimum(j + 2, NCH - 1), f0, sem0)
        compute(j + 1, f1)
        return c
    lax.fori_loop(0, NCH // 2, outer, 0)
    dma_wait(f0, sem0)

    pltpu.sync_copy(acc_v, sums_hbm.at[wid])

    @pl.when(d == 0)
    def _():
        pltpu.sync_copy(cnt_v, cnt_hbm.at[r])


def _segment_sums(cls_feats, labels):
    mesh = plsc.VectorSubcoreMesh(core_axis_name="c", subcore_axis_name="s",
                                  num_cores=NC, num_subcores=NS)
    return pl.kernel(
        _sc_body,
        out_type=(jax.ShapeDtypeStruct((NW, CP * DSUB), jnp.float32),
                  jax.ShapeDtypeStruct((NR, CP * 16), jnp.float32)),
        mesh=mesh,
        compiler_params=pltpu.CompilerParams(needs_layout_passes=False),
        scratch_types=[
            pltpu.VMEM((ROWS_R,), jnp.int32),
            pltpu.VMEM((CHUNK, 128), jnp.float32),
            pltpu.VMEM((CHUNK, 128), jnp.float32),
            pltpu.VMEM((CP * DSUB,), jnp.float32),
            pltpu.VMEM((CP * 16,), jnp.float32),
            pltpu.SemaphoreType.DMA,
            pltpu.SemaphoreType.DMA,
        ],
    )(cls_feats, labels)


def _tc_loss(sums_ref, cnt_ref, prot_ref, dp_ref, out_ref):
    parts = []
    for di in range(ND):
        s = sums_ref[0, di]
        for ri in range(1, NR):
            s = s + sums_ref[ri, di]
        parts.append(s)
    sums = jnp.concatenate(parts, axis=1)
    c16 = cnt_ref[0]
    for ri in range(1, NR):
        c16 = c16 + cnt_ref[ri]
    counts = jnp.sum(c16, axis=1, keepdims=True)
    present = counts > 0.0
    means = sums / jnp.maximum(counts, 1.0)
    delta = jnp.where(present, means, dp_ref[...])
    prot = prot_ref[...]
    an = prot / (jnp.sqrt(jnp.sum(prot * prot, axis=1, keepdims=True)) + 1e-8)
    bn = delta / (jnp.sqrt(jnp.sum(delta * delta, axis=1, keepdims=True)) + 1e-8)
    logits = lax.dot_general(an, bn, (((1,), (1,)), ((), ())),
                             preferred_element_type=jnp.float32) / TEMP
    col = lax.broadcasted_iota(jnp.int32, (CP, CP), 1)
    logits = jnp.where(col < C, logits, NEG)
    m = jnp.max(logits, axis=1, keepdims=True)
    lse = m + jnp.log(jnp.sum(jnp.exp(logits - m), axis=1, keepdims=True))
    row = lax.broadcasted_iota(jnp.int32, (CP, CP), 0)
    diag = jnp.sum(jnp.where(row == col, logits, 0.0), axis=1, keepdims=True)
    per_row = lse - diag
    pf = jnp.where(present, 1.0, 0.0)
    num = jnp.sum(per_row * pf, axis=(0, 1), keepdims=True)
    den = jnp.maximum(jnp.sum(pf, axis=(0, 1), keepdims=True), 1.0)
    out_ref[...] = num / den


def kernel(cls_feats, cls_targets, prototypes, delta_prototype):
    labels = cls_targets.reshape(N).astype(jnp.int32)
    sums_flat, cnt_flat = _segment_sums(cls_feats, labels)
    sums4d = sums_flat.reshape(NR, ND, CP, DSUB)
    cnt3d = cnt_flat.reshape(NR, CP, 16)
    prot_pad = jnp.pad(prototypes, ((0, CP - C), (0, 0)))
    dp_pad = jnp.pad(delta_prototype, ((0, CP - C), (0, 0)))
    loss = pl.pallas_call(
        _tc_loss,
        out_shape=jax.ShapeDtypeStruct((1, 1), jnp.float32),
    )(sums4d, cnt3d, prot_pad, dp_pad)
    return loss[0, 0]


# speedup vs baseline: 1.0391x; 1.0391x over previous
import jax
import jax.numpy as jnp
from jax import lax
from jax.experimental import pallas as pl
from jax.experimental.pallas import tpu as pltpu
from jax.experimental.pallas import tpu_sc as plsc

C = 1203
D = 256
N = 65536
TEMP = 0.07
CP = 1280
NC, NS = 2, 16
NW = NC * NS
ND = 4
DSUB = D // ND
NR = NW // ND
NSC = 20480
NTC = N - NSC
ROWS_R = NSC // NR
CHUNK = 64
NCH = ROWS_R // CHUNK
BLK = 2048
NEG = -1e30


def _sc_body(feats_hbm, labels_hbm, sums_hbm, cnt_hbm,
             labels_v, f0, f1, acc_v, cnt_v, sem0, sem1):
    cid = lax.axis_index("c")
    sid = lax.axis_index("s")
    wid = cid * NS + sid
    r = wid // ND
    d = wid % ND
    h = d // 2
    off = (d % 2) * DSUB
    row0 = r * ROWS_R

    zero16 = jnp.zeros((16,), jnp.float32)
    ones16 = jnp.ones((16,), jnp.float32)
    lane = lax.iota(jnp.int32, 16)
    cols = [lane + k * 16 for k in range(DSUB // 16)]

    @plsc.parallel_loop(0, CP * DSUB // 16, 1, unroll=4)
    def _zero(i):
        acc_v[pl.ds(i * 16, 16)] = zero16

    @plsc.parallel_loop(0, CP, 1, unroll=4)
    def _zeroc(i):
        cnt_v[pl.ds(i * 16, 16)] = zero16

    pltpu.sync_copy(labels_hbm.at[pl.ds(row0, ROWS_R)], labels_v)

    def dma_start(j, buf, sem):
        pltpu.async_copy(
            feats_hbm.at[pl.ds(row0 + j * CHUNK, CHUNK), pl.ds(h * 128, 128)],
            buf, sem)

    def dma_wait(buf, sem):
        pltpu.make_async_copy(
            feats_hbm.at[pl.ds(row0, CHUNK), pl.ds(h * 128, 128)],
            buf, sem).wait()

    def compute(j, buf):
        jbase = j * CHUNK

        @pl.when(d == 0)
        def _():
            @plsc.parallel_loop(0, CHUNK // 16, 1, unroll=2)
            def _cnt16(t):
                lv = labels_v[pl.ds(jbase + t * 16, 16)]
                plsc.addupdate_scatter(cnt_v, [lv * 16 + lane], ones16)

        @plsc.parallel_loop(0, CHUNK, 1, unroll=4)
        def _rows(i):
            ridx = jnp.full((16,), jbase + i, jnp.int32)
            bl = plsc.load_gather(labels_v, [ridx]) * DSUB
            for k in range(DSUB // 16):
                v = buf[i, pl.ds(off + k * 16, 16)]
                plsc.addupdate_scatter(acc_v, [bl + cols[k]], v)

    dma_start(0, f0, sem0)

    def outer(j2, c):
        j = j2 * 2
        dma_wait(f0, sem0)
        dma_start(jnp.minimum(j + 1, NCH - 1), f1, sem1)
        compute(j, f0)
        dma_wait(f1, sem1)
        dma_start(jnp.minimum(j + 2, NCH - 1), f0, sem0)
        compute(j + 1, f1)
        return c
    lax.fori_loop(0, NCH // 2, outer, 0)
    dma_wait(f0, sem0)

    pltpu.sync_copy(acc_v, sums_hbm.at[wid])

    @pl.when(d == 0)
    def _():
        pltpu.sync_copy(cnt_v, cnt_hbm.at[r])


def _segment_sums(cls_feats, labels):
    mesh = plsc.VectorSubcoreMesh(core_axis_name="c", subcore_axis_name="s",
                                  num_cores=NC, num_subcores=NS)
    return pl.kernel(
        _sc_body,
        out_type=(jax.ShapeDtypeStruct((NW, CP * DSUB), jnp.float32),
                  jax.ShapeDtypeStruct((NR, CP * 16), jnp.float32)),
        mesh=mesh,
        compiler_params=pltpu.CompilerParams(needs_layout_passes=False),
        scratch_types=[
            pltpu.VMEM((ROWS_R,), jnp.int32),
            pltpu.VMEM((CHUNK, 128), jnp.float32),
            pltpu.VMEM((CHUNK, 128), jnp.float32),
            pltpu.VMEM((CP * DSUB,), jnp.float32),
            pltpu.VMEM((CP * 16,), jnp.float32),
            pltpu.SemaphoreType.DMA,
            pltpu.SemaphoreType.DMA,
        ],
    )(cls_feats, labels)


def _tc_seg_body(feats_ref, lab_ref, s_ref, c_ref):
    x = feats_ref[...]
    xhi = x.astype(jnp.bfloat16)
    xlo = (x - xhi.astype(jnp.float32)).astype(jnp.bfloat16)
    lab = lab_ref[...]
    iota_c = lax.broadcasted_iota(jnp.int32, (CP, 128), 0)
    acc = jnp.zeros((CP, D), jnp.float32)
    cnt = jnp.zeros((CP, 1), jnp.float32)
    for a in range(BLK // 128):
        la = lab[a:a + 1, :]
        ohb = (iota_c == la)
        oh = ohb.astype(jnp.bfloat16)
        acc = acc + lax.dot_general(oh, xhi[a * 128:(a + 1) * 128, :],
                                    (((1,), (0,)), ((), ())),
                                    preferred_element_type=jnp.float32)
        acc = acc + lax.dot_general(oh, xlo[a * 128:(a + 1) * 128, :],
                                    (((1,), (0,)), ((), ())),
                                    preferred_element_type=jnp.float32)
        cnt = cnt + jnp.sum(ohb.astype(jnp.float32), axis=1, keepdims=True)

    @pl.when(pl.program_id(0) == 0)
    def _():
        s_ref[...] = jnp.zeros_like(s_ref)
        c_ref[...] = jnp.zeros_like(c_ref)

    s_ref[...] += acc
    c_ref[...] += jnp.broadcast_to(cnt, (CP, 8))


def _tc_segment(cls_feats, labels2d):
    return pl.pallas_call(
        _tc_seg_body,
        grid=(NTC // BLK,),
        in_specs=[
            pl.BlockSpec((BLK, D), lambda j: (NSC // BLK + j, 0)),
            pl.BlockSpec((BLK // 128, 128), lambda j: (NSC // BLK + j, 0)),
        ],
        out_specs=[
            pl.BlockSpec((CP, D), lambda j: (0, 0)),
            pl.BlockSpec((CP, 8), lambda j: (0, 0)),
        ],
        out_shape=[jax.ShapeDtypeStruct((CP, D), jnp.float32),
                   jax.ShapeDtypeStruct((CP, 8), jnp.float32)],
    )(cls_feats, labels2d)


def _tc_loss(sums_ref, cnt_ref, stc_ref, ctc_ref, prot_ref, dp_ref, out_ref):
    parts = []
    for di in range(ND):
        s = sums_ref[0, di]
        for ri in range(1, NR):
            s = s + sums_ref[ri, di]
        parts.append(s)
    sums = jnp.concatenate(parts, axis=1) + stc_ref[...]
    c16 = cnt_ref[0]
    for ri in range(1, NR):
        c16 = c16 + cnt_ref[ri]
    counts = jnp.sum(c16, axis=1, keepdims=True) + ctc_ref[:, 0:1]
    present = counts > 0.0
    means = sums / jnp.maximum(counts, 1.0)
    delta = jnp.where(present, means, dp_ref[...])
    prot = prot_ref[...]
    an = prot / (jnp.sqrt(jnp.sum(prot * prot, axis=1, keepdims=True)) + 1e-8)
    bn = delta / (jnp.sqrt(jnp.sum(delta * delta, axis=1, keepdims=True)) + 1e-8)
    logits = lax.dot_general(an, bn, (((1,), (1,)), ((), ())),
                             preferred_element_type=jnp.float32) / TEMP
    col = lax.broadcasted_iota(jnp.int32, (CP, CP), 1)
    logits = jnp.where(col < C, logits, NEG)
    m = jnp.max(logits, axis=1, keepdims=True)
    lse = m + jnp.log(jnp.sum(jnp.exp(logits - m), axis=1, keepdims=True))
    row = lax.broadcasted_iota(jnp.int32, (CP, CP), 0)
    diag = jnp.sum(jnp.where(row == col, logits, 0.0), axis=1, keepdims=True)
    per_row = lse - diag
    pf = jnp.where(present, 1.0, 0.0)
    num = jnp.sum(per_row * pf, axis=(0, 1), keepdims=True)
    den = jnp.maximum(jnp.sum(pf, axis=(0, 1), keepdims=True), 1.0)
    out_ref[...] = num / den


def kernel(cls_feats, cls_targets, prototypes, delta_prototype):
    labels = cls_targets.reshape(N).astype(jnp.int32)
    sums_flat, cnt_flat = _segment_sums(cls_feats, labels)
    s_tc, c_tc = _tc_segment(cls_feats, labels.reshape(N // 128, 128))
    sums4d = sums_flat.reshape(NR, ND, CP, DSUB)
    cnt3d = cnt_flat.reshape(NR, CP, 16)
    prot_pad = jnp.pad(prototypes, ((0, CP - C), (0, 0)))
    dp_pad = jnp.pad(delta_prototype, ((0, CP - C), (0, 0)))
    loss = pl.pallas_call(
        _tc_loss,
        out_shape=jax.ShapeDtypeStruct((1, 1), jnp.float32),
    )(sums4d, cnt3d, s_tc, c_tc, prot_pad, dp_pad)
    return loss[0, 0]


# speedup vs baseline: 1.1931x; 1.1482x over previous
import jax
import jax.numpy as jnp
from jax import lax
from jax.experimental import pallas as pl
from jax.experimental.pallas import tpu as pltpu
from jax.experimental.pallas import tpu_sc as plsc

C = 1203
D = 256
N = 65536
TEMP = 0.07
CP = 1280
NC, NS = 2, 16
NW = NC * NS
ND = 4
DSUB = D // ND
NR = NW // ND
ROWS_R = N // NR
CHUNK = 128
NCH = ROWS_R // CHUNK
NEG = -1e30


def _sc_body(feats_hbm, labels_hbm, sums_hbm, cnt_hbm,
             labels_v, f0, f1, acc_v, cnt_v, sem0, sem1):
    cid = lax.axis_index("c")
    sid = lax.axis_index("s")
    wid = cid * NS + sid
    r = wid // ND
    d = wid % ND
    row0 = r * ROWS_R
    col0 = d * DSUB

    zero16 = jnp.zeros((16,), jnp.float32)
    ones16 = jnp.ones((16,), jnp.float32)
    lane = lax.iota(jnp.int32, 16)
    cols = [lane + k * 16 for k in range(DSUB // 16)]

    @plsc.parallel_loop(0, CP, 1, unroll=4)
    def _zero(i):
        for k in range(DSUB // 16):
            acc_v[i, pl.ds(k * 16, 16)] = zero16
        cnt_v[i, :] = zero16

    pltpu.sync_copy(labels_hbm.at[pl.ds(row0, ROWS_R)], labels_v)

    def dma_start(j, buf, sem):
        pltpu.async_copy(
            feats_hbm.at[pl.ds(row0 + j * CHUNK, CHUNK), pl.ds(col0, DSUB)],
            buf, sem)

    def dma_wait(buf, sem):
        pltpu.make_async_copy(
            feats_hbm.at[pl.ds(row0, CHUNK), pl.ds(col0, DSUB)],
            buf, sem).wait()

    def compute(j, buf):
        jbase = j * CHUNK

        @pl.when(d == 0)
        def _():
            @plsc.parallel_loop(0, CHUNK // 16, 1, unroll=2)
            def _cnt16(t):
                lv = labels_v[pl.ds(jbase + t * 16, 16)]
                plsc.addupdate_scatter(cnt_v, [lv, lane], ones16)

        @plsc.parallel_loop(0, CHUNK, 1, unroll=8)
        def _rows(i):
            ridx = jnp.full((16,), jbase + i, jnp.int32)
            bl = plsc.load_gather(labels_v, [ridx])
            for k in range(DSUB // 16):
                v = buf[i, pl.ds(k * 16, 16)]
                plsc.addupdate_scatter(acc_v, [bl, cols[k]], v)

    dma_start(0, f0, sem0)

    def outer(j2, c):
        j = j2 * 2
        dma_wait(f0, sem0)
        dma_start(jnp.minimum(j + 1, NCH - 1), f1, sem1)
        compute(j, f0)
        dma_wait(f1, sem1)
        dma_start(jnp.minimum(j + 2, NCH - 1), f0, sem0)
        compute(j + 1, f1)
        return c
    lax.fori_loop(0, NCH // 2, outer, 0)
    dma_wait(f0, sem0)

    pltpu.sync_copy(acc_v, sums_hbm.at[r, :, pl.ds(col0, DSUB)])

    @pl.when(d == 0)
    def _():
        pltpu.sync_copy(cnt_v, cnt_hbm.at[r])


def _segment_sums(cls_feats, labels):
    mesh = plsc.VectorSubcoreMesh(core_axis_name="c", subcore_axis_name="s",
                                  num_cores=NC, num_subcores=NS)
    return pl.kernel(
        _sc_body,
        out_type=(jax.ShapeDtypeStruct((NR, CP, D), jnp.float32),
                  jax.ShapeDtypeStruct((NR, CP, 16), jnp.float32)),
        mesh=mesh,
        compiler_params=pltpu.CompilerParams(use_tc_tiling_on_sc=False,
                                             needs_layout_passes=False),
        scratch_types=[
            pltpu.VMEM((ROWS_R,), jnp.int32),
            pltpu.VMEM((CHUNK, DSUB), jnp.float32),
            pltpu.VMEM((CHUNK, DSUB), jnp.float32),
            pltpu.VMEM((CP, DSUB), jnp.float32),
            pltpu.VMEM((CP, 16), jnp.float32),
            pltpu.SemaphoreType.DMA,
            pltpu.SemaphoreType.DMA,
        ],
    )(cls_feats, labels)


def _tc_loss(sums_ref, cnt_ref, prot_ref, dp_ref, out_ref):
    sums = sums_ref[0]
    for i in range(1, NR):
        sums = sums + sums_ref[i]
    c16 = cnt_ref[0]
    for i in range(1, NR):
        c16 = c16 + cnt_ref[i]
    counts = jnp.sum(c16, axis=1, keepdims=True)
    present = counts > 0.0
    means = sums / jnp.maximum(counts, 1.0)
    delta = jnp.where(present, means, dp_ref[...])
    prot = prot_ref[...]
    an = prot / (jnp.sqrt(jnp.sum(prot * prot, axis=1, keepdims=True)) + 1e-8)
    bn = delta / (jnp.sqrt(jnp.sum(delta * delta, axis=1, keepdims=True)) + 1e-8)
    logits = lax.dot_general(an, bn, (((1,), (1,)), ((), ())),
                             preferred_element_type=jnp.float32) / TEMP
    col = lax.broadcasted_iota(jnp.int32, (CP, CP), 1)
    logits = jnp.where(col < C, logits, NEG)
    m = jnp.max(logits, axis=1, keepdims=True)
    lse = m + jnp.log(jnp.sum(jnp.exp(logits - m), axis=1, keepdims=True))
    row = lax.broadcasted_iota(jnp.int32, (CP, CP), 0)
    diag = jnp.sum(jnp.where(row == col, logits, 0.0), axis=1, keepdims=True)
    per_row = lse - diag
    pf = jnp.where(present, 1.0, 0.0)
    num = jnp.sum(per_row * pf, axis=(0, 1), keepdims=True)
    den = jnp.maximum(jnp.sum(pf, axis=(0, 1), keepdims=True), 1.0)
    out_ref[...] = num / den


def kernel(cls_feats, cls_targets, prototypes, delta_prototype):
    labels = cls_targets.reshape(N).astype(jnp.int32)
    sums8, cnt8 = _segment_sums(cls_feats, labels)
    prot_pad = jnp.pad(prototypes, ((0, CP - C), (0, 0)))
    dp_pad = jnp.pad(delta_prototype, ((0, CP - C), (0, 0)))
    loss = pl.pallas_call(
        _tc_loss,
        out_shape=jax.ShapeDtypeStruct((1, 1), jnp.float32),
    )(sums8, cnt8, prot_pad, dp_pad)
    return loss[0, 0]


# speedup vs baseline: 1.3175x; 1.1043x over previous
import jax
import jax.numpy as jnp
from jax import lax
from jax.experimental import pallas as pl
from jax.experimental.pallas import tpu as pltpu
from jax.experimental.pallas import tpu_sc as plsc

C = 1203
D = 256
N = 65536
TEMP = 0.07
CP = 1280
NC, NS = 2, 16
NW = NC * NS
ND = 4
DSUB = D // ND
NR = NW // ND
ROWS_R = N // NR
CHUNK = 64
NCH = ROWS_R // CHUNK
NBUF = 4
NEG = -1e30


def _sc_body(feats_hbm, labels_hbm, sums_hbm, cnt_hbm,
             labels_v, f0, f1, f2, f3, acc_v, cnt_v, sem0, sem1, sem2, sem3):
    cid = lax.axis_index("c")
    sid = lax.axis_index("s")
    wid = cid * NS + sid
    r = wid // ND
    d = wid % ND
    row0 = r * ROWS_R
    col0 = d * DSUB

    zero16 = jnp.zeros((16,), jnp.float32)
    ones16 = jnp.ones((16,), jnp.float32)
    lane = lax.iota(jnp.int32, 16)
    cols = [lane + k * 16 for k in range(DSUB // 16)]

    @plsc.parallel_loop(0, CP, 1, unroll=4)
    def _zero(i):
        for k in range(DSUB // 16):
            acc_v[i, pl.ds(k * 16, 16)] = zero16
        cnt_v[i, :] = zero16

    pltpu.sync_copy(labels_hbm.at[pl.ds(row0, ROWS_R)], labels_v)

    def dma_start(j, buf, sem):
        pltpu.async_copy(
            feats_hbm.at[pl.ds(row0 + j * CHUNK, CHUNK), pl.ds(col0, DSUB)],
            buf, sem)

    def dma_wait(buf, sem):
        pltpu.make_async_copy(
            feats_hbm.at[pl.ds(row0, CHUNK), pl.ds(col0, DSUB)],
            buf, sem).wait()

    def compute(j, buf):
        jbase = j * CHUNK

        @pl.when(d == 0)
        def _():
            @plsc.parallel_loop(0, CHUNK // 16, 1, unroll=2)
            def _cnt16(t):
                lv = labels_v[pl.ds(jbase + t * 16, 16)]
                plsc.addupdate_scatter(cnt_v, [lv, lane], ones16)

        @plsc.parallel_loop(0, CHUNK, 1, unroll=8)
        def _rows(i):
            ridx = jnp.full((16,), jbase + i, jnp.int32)
            bl = plsc.load_gather(labels_v, [ridx])
            for k in range(DSUB // 16):
                v = buf[i, pl.ds(k * 16, 16)]
                plsc.addupdate_scatter(acc_v, [bl, cols[k]], v)

    bufs = (f0, f1, f2, f3)
    sems = (sem0, sem1, sem2, sem3)
    for b in range(NBUF - 1):
        dma_start(b, bufs[b], sems[b])

    def outer(jj, c):
        j = jj * NBUF
        for b in range(NBUF):
            dma_wait(bufs[b], sems[b])
            nb = (b + NBUF - 1) % NBUF
            dma_start(jnp.minimum(j + b + NBUF - 1, NCH - 1), bufs[nb], sems[nb])
            compute(j + b, bufs[b])
        return c
    lax.fori_loop(0, NCH // NBUF, outer, 0)
    for b in range(NBUF - 1):
        dma_wait(bufs[b], sems[b])

    pltpu.sync_copy(acc_v, sums_hbm.at[r, :, pl.ds(col0, DSUB)])

    @pl.when(d == 0)
    def _():
        pltpu.sync_copy(cnt_v, cnt_hbm.at[r])


def _segment_sums(cls_feats, labels):
    mesh = plsc.VectorSubcoreMesh(core_axis_name="c", subcore_axis_name="s",
                                  num_cores=NC, num_subcores=NS)
    return pl.kernel(
        _sc_body,
        out_type=(jax.ShapeDtypeStruct((NR, CP, D), jnp.float32),
                  jax.ShapeDtypeStruct((NR, CP, 16), jnp.float32)),
        mesh=mesh,
        compiler_params=pltpu.CompilerParams(use_tc_tiling_on_sc=False,
                                             needs_layout_passes=False),
        scratch_types=[
            pltpu.VMEM((ROWS_R,), jnp.int32),
            pltpu.VMEM((CHUNK, DSUB), jnp.float32),
            pltpu.VMEM((CHUNK, DSUB), jnp.float32),
            pltpu.VMEM((CHUNK, DSUB), jnp.float32),
            pltpu.VMEM((CHUNK, DSUB), jnp.float32),
            pltpu.VMEM((CP, DSUB), jnp.float32),
            pltpu.VMEM((CP, 16), jnp.float32),
            pltpu.SemaphoreType.DMA,
            pltpu.SemaphoreType.DMA,
            pltpu.SemaphoreType.DMA,
            pltpu.SemaphoreType.DMA,
        ],
    )(cls_feats, labels)


def _tc_loss(sums_ref, cnt_ref, prot_ref, dp_ref, out_ref):
    sums = sums_ref[0]
    for i in range(1, NR):
        sums = sums + sums_ref[i]
    c16 = cnt_ref[0]
    for i in range(1, NR):
        c16 = c16 + cnt_ref[i]
    counts = jnp.sum(c16, axis=1, keepdims=True)
    present = counts > 0.0
    means = sums / jnp.maximum(counts, 1.0)
    delta = jnp.where(present, means, dp_ref[...])
    prot = prot_ref[...]
    an = prot / (jnp.sqrt(jnp.sum(prot * prot, axis=1, keepdims=True)) + 1e-8)
    bn = delta / (jnp.sqrt(jnp.sum(delta * delta, axis=1, keepdims=True)) + 1e-8)
    logits = lax.dot_general(an, bn, (((1,), (1,)), ((), ())),
                             preferred_element_type=jnp.float32) / TEMP
    col = lax.broadcasted_iota(jnp.int32, (CP, CP), 1)
    logits = jnp.where(col < C, logits, NEG)
    m = jnp.max(logits, axis=1, keepdims=True)
    lse = m + jnp.log(jnp.sum(jnp.exp(logits - m), axis=1, keepdims=True))
    row = lax.broadcasted_iota(jnp.int32, (CP, CP), 0)
    diag = jnp.sum(jnp.where(row == col, logits, 0.0), axis=1, keepdims=True)
    per_row = lse - diag
    pf = jnp.where(present, 1.0, 0.0)
    num = jnp.sum(per_row * pf, axis=(0, 1), keepdims=True)
    den = jnp.maximum(jnp.sum(pf, axis=(0, 1), keepdims=True), 1.0)
    out_ref[...] = num / den


def kernel(cls_feats, cls_targets, prototypes, delta_prototype):
    labels = cls_targets.reshape(N).astype(jnp.int32)
    sums8, cnt8 = _segment_sums(cls_feats, labels)
    prot_pad = jnp.pad(prototypes, ((0, CP - C), (0, 0)))
    dp_pad = jnp.pad(delta_prototype, ((0, CP - C), (0, 0)))
    loss = pl.pallas_call(
        _tc_loss,
        out_shape=jax.ShapeDtypeStruct((1, 1), jnp.float32),
    )(sums8, cnt8, prot_pad, dp_pad)
    return loss[0, 0]
